# baseline (speedup 1.0000x reference)
import functools

import jax
import jax.numpy as jnp
from jax.experimental import pallas as pl

_DIM = 2048
_NE = 64
_K = 8
_BT = 512


def _gate_block(x_ref, w_ref, vals_ref, idx_ref):
    x = x_ref[...]
    w = w_ref[...]
    logits = jax.lax.dot_general(
        x, w, (((1,), (1,)), ((), ())), preferred_element_type=jnp.float32
    )
    iota_col = jax.lax.broadcasted_iota(jnp.int32, (_NE, 1), 0).astype(jnp.float32)
    neg_inf = jnp.float32(-jnp.inf)
    vals = []
    onehots = []
    l = logits
    for _ in range(_K):
        m = jnp.max(l, axis=1, keepdims=True)
        is_m = l == m
        l = jnp.where(is_m, neg_inf, l)
        vals.append(m)
        onehots.append(is_m)
    oh = jnp.concatenate(onehots, axis=0).astype(jnp.float32)
    idx_f = jax.lax.dot_general(
        oh, iota_col, (((1,), (0,)), ((), ())), preferred_element_type=jnp.float32
    )
    top_idx = jnp.concatenate(
        [idx_f[j * _BT : (j + 1) * _BT] for j in range(_K)], axis=1
    ).astype(jnp.int32)
    top = jnp.concatenate(vals, axis=1)
    s = jax.nn.sigmoid(top)
    s = s / jnp.sum(s, axis=1, keepdims=True)
    vals_ref[...] = s
    idx_ref[...] = top_idx


@jax.jit
def kernel(x, weight):
    t = x.shape[0]
    grid = (t // _BT,)
    vals, idx = pl.pallas_call(
        _gate_block,
        grid=grid,
        in_specs=[
            pl.BlockSpec((_BT, _DIM), lambda i: (i, 0)),
            pl.BlockSpec((_NE, _DIM), lambda i: (0, 0)),
        ],
        out_specs=[
            pl.BlockSpec((_BT, _K), lambda i: (i, 0)),
            pl.BlockSpec((_BT, _K), lambda i: (i, 0)),
        ],
        out_shape=[
            jax.ShapeDtypeStruct((t, _K), jnp.float32),
            jax.ShapeDtypeStruct((t, _K), jnp.int32),
        ],
    )(x, weight)
    return vals, idx


# speedup vs baseline: 2.1235x; 2.1235x over previous
import functools

import jax
import jax.numpy as jnp
from jax.experimental import pallas as pl

_DIM = 2048
_NE = 64
_K = 8
_BT = 512


def _gate_block(x_ref, w_ref, vals_ref, idx_ref):
    x = x_ref[...]
    w = w_ref[...]
    logits = jax.lax.dot_general(
        x, w, (((1,), (1,)), ((), ())), preferred_element_type=jnp.float32
    )
    lanes = jax.lax.broadcasted_iota(jnp.int32, (_BT, _NE), 1)
    bits = jax.lax.bitcast_convert_type(logits, jnp.int32)
    low = jnp.where(bits < 0, lanes, (_NE - 1) - lanes)
    keys = jax.lax.bitcast_convert_type((bits & ~(_NE - 1)) | low, jnp.float32)
    neg_inf = jnp.float32(-jnp.inf)
    tops = []
    for _ in range(_K):
        m = jnp.max(keys, axis=1, keepdims=True)
        keys = jnp.where(keys == m, neg_inf, keys)
        tops.append(m)
    top = jnp.concatenate(tops, axis=1)
    tb = jax.lax.bitcast_convert_type(top, jnp.int32)
    low6 = tb & (_NE - 1)
    top_idx = jnp.where(tb < 0, low6, (_NE - 1) - low6)
    top_val = jax.lax.bitcast_convert_type(tb & ~(_NE - 1), jnp.float32)
    s = jax.nn.sigmoid(top_val)
    s = s / jnp.sum(s, axis=1, keepdims=True)
    vals_ref[...] = s
    idx_ref[...] = top_idx


@jax.jit
def kernel(x, weight):
    t = x.shape[0]
    grid = (t // _BT,)
    vals, idx = pl.pallas_call(
        _gate_block,
        grid=grid,
        in_specs=[
            pl.BlockSpec((_BT, _DIM), lambda i: (i, 0)),
            pl.BlockSpec((_NE, _DIM), lambda i: (0, 0)),
        ],
        out_specs=[
            pl.BlockSpec((_BT, _K), lambda i: (i, 0)),
            pl.BlockSpec((_BT, _K), lambda i: (i, 0)),
        ],
        out_shape=[
            jax.ShapeDtypeStruct((t, _K), jnp.float32),
            jax.ShapeDtypeStruct((t, _K), jnp.int32),
        ],
    )(x, weight)
    return vals, idx


# speedup vs baseline: 2.4963x; 1.1756x over previous
import functools

import jax
import jax.numpy as jnp
from jax.experimental import pallas as pl

_DIM = 2048
_NE = 64
_K = 8
_BT = 1024
_S = 8
_SW = _DIM // _S


def _gate_block(*refs):
    x_refs = refs[:_S]
    w_ref = refs[_S]
    vals_ref, idx_ref = refs[_S + 1], refs[_S + 2]
    w = w_ref[...]
    logits = jnp.zeros((_BT, _NE), jnp.float32)
    for s in range(_S):
        logits += jax.lax.dot_general(
            x_refs[s][...],
            w[:, s * _SW : (s + 1) * _SW],
            (((1,), (1,)), ((), ())),
            preferred_element_type=jnp.float32,
        )
    lanes = jax.lax.broadcasted_iota(jnp.int32, (_BT, _NE), 1)
    bits = jax.lax.bitcast_convert_type(logits, jnp.int32)
    low = jnp.where(bits < 0, lanes, (_NE - 1) - lanes)
    keys = jax.lax.bitcast_convert_type((bits & ~(_NE - 1)) | low, jnp.float32)
    neg_inf = jnp.float32(-jnp.inf)
    tops = []
    for _ in range(_K):
        m = jnp.max(keys, axis=1, keepdims=True)
        keys = jnp.where(keys == m, neg_inf, keys)
        tops.append(m)
    top = jnp.concatenate(tops, axis=1)
    tb = jax.lax.bitcast_convert_type(top, jnp.int32)
    low6 = tb & (_NE - 1)
    top_idx = jnp.where(tb < 0, low6, (_NE - 1) - low6)
    top_val = jax.lax.bitcast_convert_type(tb & ~(_NE - 1), jnp.float32)
    s = jax.nn.sigmoid(top_val)
    s = s / jnp.sum(s, axis=1, keepdims=True)
    vals_ref[...] = s
    idx_ref[...] = top_idx


@jax.jit
def kernel(x, weight):
    t = x.shape[0]
    grid = (t // _BT,)
    vals, idx = pl.pallas_call(
        _gate_block,
        grid=grid,
        in_specs=[
            pl.BlockSpec((_BT, _SW), functools.partial(lambda s, i: (i, s), s))
            for s in range(_S)
        ]
        + [
            pl.BlockSpec((_NE, _DIM), lambda i: (0, 0)),
        ],
        out_specs=[
            pl.BlockSpec((_BT, _K), lambda i: (i, 0)),
            pl.BlockSpec((_BT, _K), lambda i: (i, 0)),
        ],
        out_shape=[
            jax.ShapeDtypeStruct((t, _K), jnp.float32),
            jax.ShapeDtypeStruct((t, _K), jnp.int32),
        ],
    )(*([x] * _S), weight)
    return vals, idx
